# baseline (speedup 1.0000x reference)
import jax
import jax.numpy as jnp
from jax.experimental import pallas as pl

POP_ = 128
GENE_ = 4096
BJ_ = 256

def _quadform_kernel(x_ref, q_ref, out_ref):
    j = pl.program_id(0)
    x = x_ref[...]
    q = q_ref[...]
    xj = x_ref[:, pl.ds(j * BJ_, BJ_)]
    y = jnp.dot(xj, q, preferred_element_type=jnp.float32,
                precision=jax.lax.Precision.DEFAULT)
    partial = jnp.sum(y * x, axis=1)
    @pl.when(j == 0)
    def _init():
        out_ref[...] = partial[None, :]
    @pl.when(j > 0)
    def _acc():
        out_ref[...] += partial[None, :]

@jax.jit
def kernel(keys_pop, Q):
    out = pl.pallas_call(
        _quadform_kernel,
        grid=(GENE_ // BJ_,),
        in_specs=[
            pl.BlockSpec((POP_, GENE_), lambda j: (0, 0)),
            pl.BlockSpec((BJ_, GENE_), lambda j: (j, 0)),
        ],
        out_specs=pl.BlockSpec((1, POP_), lambda j: (0, 0)),
        out_shape=jax.ShapeDtypeStruct((1, POP_), jnp.float32),
    )(keys_pop, Q)
    return out[0]


# speedup vs baseline: 1.1807x; 1.1807x over previous
import jax
import jax.numpy as jnp
from jax.experimental import pallas as pl
from jax.experimental.pallas import tpu as pltpu

POP_ = 128
GENE_ = 4096
BK_ = 512
NK_ = GENE_ // BK_


def _quadform_kernel(x_ref, q_ref, out_ref, acc_ref):
    k = pl.program_id(0)
    x = x_ref[...]
    q = q_ref[...]
    y = jnp.dot(x, q, preferred_element_type=jnp.float32,
                precision=jax.lax.Precision.DEFAULT)
    xk = x_ref[:, pl.ds(k * BK_, BK_)]
    z = y * xk

    @pl.when(k == 0)
    def _init():
        acc_ref[...] = z

    @pl.when(k > 0)
    def _acc():
        acc_ref[...] += z

    @pl.when(k == NK_ - 1)
    def _finish():
        out_ref[...] = jnp.sum(acc_ref[...], axis=1)[None, :]


@jax.jit
def kernel(keys_pop, Q):
    out = pl.pallas_call(
        _quadform_kernel,
        grid=(NK_,),
        in_specs=[
            pl.BlockSpec((POP_, GENE_), lambda k: (0, 0)),
            pl.BlockSpec((GENE_, BK_), lambda k: (0, k)),
        ],
        out_specs=pl.BlockSpec((1, POP_), lambda k: (0, 0)),
        out_shape=jax.ShapeDtypeStruct((1, POP_), jnp.float32),
        scratch_shapes=[pltpu.VMEM((POP_, BK_), jnp.float32)],
    )(keys_pop, Q)
    return out[0]
